# baseline (speedup 1.0000x reference)
import functools

import jax
import jax.numpy as jnp
from jax import lax
from jax.experimental import pallas as pl
from jax.experimental.pallas import tpu as pltpu
from jax.experimental.pallas import tpu_sc as plsc


def kernel(x, table):
    seq_len = x.shape[1]
    embed_dim = table.shape[1]
    info = plsc.get_sparse_core_info()
    num_workers = info.num_cores * info.num_subcores
    assert seq_len % num_workers == 0
    rows_per_worker = seq_len // num_workers
    mesh = plsc.VectorSubcoreMesh(core_axis_name="c", subcore_axis_name="s")

    @functools.partial(
        pl.kernel,
        mesh=mesh,
        out_type=jax.ShapeDtypeStruct((seq_len, embed_dim), table.dtype),
    )
    def _copy(table_hbm, out_hbm):
        wid = lax.axis_index("s") * info.num_cores + lax.axis_index("c")
        base = wid * rows_per_worker
        pltpu.sync_copy(
            table_hbm.at[pl.ds(base, rows_per_worker)],
            out_hbm.at[pl.ds(base, rows_per_worker)],
        )

    return _copy(table)


# speedup vs baseline: 1.0537x; 1.0537x over previous
import jax
import jax.numpy as jnp
from jax.experimental import pallas as pl
from jax.experimental.pallas import tpu as pltpu


def _dma_copy_kernel(table_ref, out_ref, sem):
    seq_len = out_ref.shape[0]
    copy = pltpu.make_async_copy(table_ref.at[pl.ds(0, seq_len)], out_ref, sem)
    copy.start()
    copy.wait()


def kernel(x, table):
    seq_len = x.shape[1]
    embed_dim = table.shape[1]
    return pl.pallas_call(
        _dma_copy_kernel,
        in_specs=[pl.BlockSpec(memory_space=pltpu.HBM)],
        out_specs=pl.BlockSpec(memory_space=pltpu.HBM),
        out_shape=jax.ShapeDtypeStruct((seq_len, embed_dim), table.dtype),
        scratch_shapes=[pltpu.SemaphoreType.DMA],
    )(table)


# speedup vs baseline: 7.9394x; 7.5346x over previous
import jax
import jax.numpy as jnp
from jax.experimental import pallas as pl

ROW_BLOCK = 512


def _copy_kernel(table_ref, out_ref):
    out_ref[...] = table_ref[...]


def kernel(x, table):
    seq_len = x.shape[1]
    embed_dim = table.shape[1]
    assert seq_len % ROW_BLOCK == 0
    grid = (seq_len // ROW_BLOCK,)
    return pl.pallas_call(
        _copy_kernel,
        grid=grid,
        in_specs=[pl.BlockSpec((ROW_BLOCK, embed_dim), lambda i: (i, 0))],
        out_specs=pl.BlockSpec((ROW_BLOCK, embed_dim), lambda i: (i, 0)),
        out_shape=jax.ShapeDtypeStruct((seq_len, embed_dim), table.dtype),
    )(table)


# speedup vs baseline: 9.3541x; 1.1782x over previous
import jax
import jax.numpy as jnp
from jax.experimental import pallas as pl

ROW_BLOCK = 2048


def _copy_kernel(table_ref, out_ref):
    out_ref[...] = table_ref[...]


def kernel(x, table):
    seq_len = x.shape[1]
    embed_dim = table.shape[1]
    assert seq_len % ROW_BLOCK == 0
    grid = (seq_len // ROW_BLOCK,)
    return pl.pallas_call(
        _copy_kernel,
        grid=grid,
        in_specs=[pl.BlockSpec((ROW_BLOCK, embed_dim), lambda i: (i, 0))],
        out_specs=pl.BlockSpec((ROW_BLOCK, embed_dim), lambda i: (i, 0)),
        out_shape=jax.ShapeDtypeStruct((seq_len, embed_dim), table.dtype),
    )(table)


# speedup vs baseline: 9.6829x; 1.0352x over previous
import jax
import jax.numpy as jnp
from jax.experimental import pallas as pl

ROW_BLOCK = 4096


def _copy_kernel(table_ref, out_ref):
    out_ref[...] = table_ref[...]


def kernel(x, table):
    seq_len = x.shape[1]
    embed_dim = table.shape[1]
    assert seq_len % ROW_BLOCK == 0
    grid = (seq_len // ROW_BLOCK,)
    return pl.pallas_call(
        _copy_kernel,
        grid=grid,
        in_specs=[pl.BlockSpec((ROW_BLOCK, embed_dim), lambda i: (i, 0))],
        out_specs=pl.BlockSpec((ROW_BLOCK, embed_dim), lambda i: (i, 0)),
        out_shape=jax.ShapeDtypeStruct((seq_len, embed_dim), table.dtype),
    )(table)
